# Initial kernel scaffold; baseline (speedup 1.0000x reference)
import jax
import jax.numpy as jnp
from jax.experimental import pallas as pl


def kernel(x, params, edge_index):
    raise NotImplementedError("write your pallas kernel here")


# speedup vs baseline: 3.6338x; 3.6338x over previous
import jax
import jax.numpy as jnp
from jax import lax
from jax.experimental import pallas as pl
from jax.experimental.pallas import tpu as pltpu
from jax.experimental.pallas import tpu_sc as plsc

N_NODES = 10000
N_EDGES = 320000
D_FEAT = 128
N_HIDDEN = 96
HP = 128

_NC = 2
_NS = 16
_NW = _NC * _NS
_EPT = N_EDGES // _NW
_CH = 400
_NCHUNK = _EPT // _CH
_CHS = 200
_NCHUNKS = _EPT // _CHS

_EDGE_BLK = 3200
_NODE_BLK = 2000


def _sc_mesh():
    return plsc.VectorSubcoreMesh(
        core_axis_name="c", subcore_axis_name="s",
        num_cores=_NC, num_subcores=_NS)


def _pad_cols(w):
    pad = [(0, 0)] * (w.ndim - 1) + [(0, HP - w.shape[-1])]
    return jnp.pad(w, pad)


def _gather_body(a_hbm, b_hbm, src_hbm, dst_hbm, ea_hbm, eb_hbm,
                 idx_s, idx_d, rows_a, rows_b, sem_a, sem_b):
    c = lax.axis_index("c")
    s = lax.axis_index("s")
    wid = s * _NC + c
    base0 = wid * _EPT

    def chunk(i, carry):
        base = base0 + i * _CH
        pltpu.sync_copy(src_hbm.at[pl.ds(base, _CH)], idx_s)
        pltpu.sync_copy(dst_hbm.at[pl.ds(base, _CH)], idx_d)
        cp_a = pltpu.async_copy(a_hbm.at[idx_s], rows_a, sem_a)
        cp_b = pltpu.async_copy(b_hbm.at[idx_d], rows_b, sem_b)
        cp_a.wait()
        cp_b.wait()
        pltpu.sync_copy(rows_a, ea_hbm.at[pl.ds(base, _CH)])
        pltpu.sync_copy(rows_b, eb_hbm.at[pl.ds(base, _CH)])
        return carry

    lax.fori_loop(0, _NCHUNK, chunk, 0)


@jax.jit
def _sc_gather(a, b, src, dst):
    f = pl.kernel(
        _gather_body,
        out_type=[jax.ShapeDtypeStruct((N_EDGES, HP), jnp.float32),
                  jax.ShapeDtypeStruct((N_EDGES, HP), jnp.float32)],
        mesh=_sc_mesh(),
        scratch_types=[
            pltpu.VMEM((_CH,), jnp.int32),
            pltpu.VMEM((_CH,), jnp.int32),
            pltpu.VMEM((_CH, HP), jnp.float32),
            pltpu.VMEM((_CH, HP), jnp.float32),
            pltpu.SemaphoreType.DMA,
            pltpu.SemaphoreType.DMA,
        ],
    )
    return f(a, b, src, dst)


def _scatter_body(m_hbm, dst_hbm, zeros_hbm, out_hbm,
                  idx, mbuf, acc):
    c = lax.axis_index("c")
    s = lax.axis_index("s")
    wid = s * _NC + c
    base0 = wid * _EPT

    @pl.when(s == 0)
    def _zero():
        pltpu.sync_copy(zeros_hbm, acc)

    plsc.subcore_barrier()

    def chunk(i, carry):
        base = base0 + i * _CHS
        pltpu.sync_copy(dst_hbm.at[pl.ds(base, _CHS)], idx)
        pltpu.sync_copy(m_hbm.at[pl.ds(base, _CHS)], mbuf)
        pltpu.sync_copy(mbuf, acc.at[idx], add=True)
        return carry

    lax.fori_loop(0, _NCHUNKS, chunk, 0)

    plsc.subcore_barrier()

    @pl.when(s == 0)
    def _writeback():
        pltpu.sync_copy(acc, out_hbm.at[pl.ds(c * N_NODES, N_NODES)])


@jax.jit
def _sc_scatter(m, dst, zeros):
    f = pl.kernel(
        _scatter_body,
        out_type=jax.ShapeDtypeStruct((2 * N_NODES, HP), jnp.float32),
        mesh=_sc_mesh(),
        scratch_types=[
            pltpu.VMEM((_CHS,), jnp.int32),
            pltpu.VMEM((_CHS, HP), jnp.float32),
            pltpu.VMEM_SHARED((N_NODES, HP), jnp.float32),
        ],
    )
    return f(m, dst, zeros)


def _dot(x, w):
    return jax.lax.dot_general(x, w, (((1,), (0,)), ((), ())),
                               preferred_element_type=jnp.float32)


def _pre_body(x_ref, w0, b0, w1, b1, w2, b2, w3, b3, wa, ba, wb,
              h_ref, a_ref, bo_ref):
    h = jnp.maximum(_dot(x_ref[...], w0[...]) + b0[...], 0.0)
    h = jnp.maximum(_dot(h, w1[...]) + b1[...], 0.0)
    h = jnp.maximum(_dot(h, w2[...]) + b2[...], 0.0)
    h = _dot(h, w3[...]) + b3[...]
    h_ref[...] = h
    a_ref[...] = _dot(h, wa[...]) + ba[...]
    bo_ref[...] = _dot(h, wb[...])


@jax.jit
def _tc_pre(x, pre, wa, ba, wb):
    nblk = N_NODES // _NODE_BLK
    row = lambda i: (i, 0)
    cst = lambda i: (0, 0)
    ws = []
    specs = [pl.BlockSpec((_NODE_BLK, D_FEAT), row)]
    for (w, b) in pre:
        ws += [w, b.reshape(1, -1)]
        specs += [pl.BlockSpec(w.shape, cst), pl.BlockSpec((1, w.shape[1]), cst)]
    ws += [wa, ba.reshape(1, -1), wb]
    specs += [pl.BlockSpec(wa.shape, cst), pl.BlockSpec((1, HP), cst),
              pl.BlockSpec(wb.shape, cst)]
    return pl.pallas_call(
        _pre_body,
        grid=(nblk,),
        in_specs=specs,
        out_specs=[pl.BlockSpec((_NODE_BLK, N_HIDDEN), row),
                   pl.BlockSpec((_NODE_BLK, HP), row),
                   pl.BlockSpec((_NODE_BLK, HP), row)],
        out_shape=[jax.ShapeDtypeStruct((N_NODES, N_HIDDEN), jnp.float32),
                   jax.ShapeDtypeStruct((N_NODES, HP), jnp.float32),
                   jax.ShapeDtypeStruct((N_NODES, HP), jnp.float32)],
    )(x, *ws)


def _edge_body(ea_ref, eb_ref, w2, b2, w3, b3, w4, b4, m_ref):
    e = jnp.maximum(ea_ref[...] + eb_ref[...], 0.0)
    e = jnp.maximum(_dot(e, w2[...]) + b2[...], 0.0)
    e = jnp.maximum(_dot(e, w3[...]) + b3[...], 0.0)
    m_ref[...] = _dot(e, w4[...]) + b4[...]


@jax.jit
def _tc_edge(ea, eb, w2p, b2, w3, b3, w4p, b4p):
    nblk = N_EDGES // _EDGE_BLK
    row = lambda i: (i, 0)
    cst = lambda i: (0, 0)
    ws = [w2p, b2.reshape(1, -1), w3, b3.reshape(1, -1),
          w4p, b4p.reshape(1, -1)]
    specs = [pl.BlockSpec((_EDGE_BLK, HP), row)] * 2
    for w in ws:
        specs.append(pl.BlockSpec(w.shape, cst))
    return pl.pallas_call(
        _edge_body,
        grid=(nblk,),
        in_specs=specs,
        out_specs=pl.BlockSpec((_EDGE_BLK, HP), row),
        out_shape=jax.ShapeDtypeStruct((N_EDGES, HP), jnp.float32),
    )(ea, eb, *ws)


def _node_call(h, parts, node, heads, out_dims, final):
    p0 = parts[:N_NODES]
    p1 = parts[N_NODES:]
    nblk = N_NODES // _NODE_BLK
    row = lambda i: (i, 0)
    cst = lambda i: (0, 0)
    (wn1, bn1), n2, n3, n4 = node
    wna = wn1[:N_HIDDEN]
    wnb = jnp.pad(wn1[N_HIDDEN:], ((0, HP - N_HIDDEN), (0, 0)))
    ws = [wna, wnb, bn1.reshape(1, -1)]
    specs = [pl.BlockSpec((_NODE_BLK, N_HIDDEN), row),
             pl.BlockSpec((_NODE_BLK, HP), row),
             pl.BlockSpec((_NODE_BLK, HP), row)]
    specs += [pl.BlockSpec(wna.shape, cst), pl.BlockSpec(wnb.shape, cst),
              pl.BlockSpec((1, N_HIDDEN), cst)]
    for (w, b) in (n2, n3, n4):
        ws += [w, b.reshape(1, -1)]
        specs += [pl.BlockSpec(w.shape, cst), pl.BlockSpec((1, w.shape[1]), cst)]
    for hmat in heads:
        shp = hmat.shape if hmat.ndim == 2 else (1, hmat.shape[-1])
        ws.append(hmat.reshape(shp))
        specs.append(pl.BlockSpec(shp, cst))

    def body(h_ref, p0_ref, p1_ref, rwna, rwnb, rbn1, rw2, rb2, rw3, rb3,
             rw4, rb4, *rest):
        agg = p0_ref[...] + p1_ref[...]
        e = jnp.maximum(_dot(h_ref[...], rwna[...]) + _dot(agg, rwnb[...])
                        + rbn1[...], 0.0)
        e = jnp.maximum(_dot(e, rw2[...]) + rb2[...], 0.0)
        e = jnp.maximum(_dot(e, rw3[...]) + rb3[...], 0.0)
        hn = _dot(e, rw4[...]) + rb4[...]
        if final:
            rwo, rbo, out_ref = rest
            out_ref[...] = _dot(hn, rwo[...]) + rbo[...]
        else:
            rwa, rba, rwb, h_out, a_out, b_out = rest
            h_out[...] = hn
            a_out[...] = _dot(hn, rwa[...]) + rba[...]
            b_out[...] = _dot(hn, rwb[...])

    return pl.pallas_call(
        body,
        grid=(nblk,),
        in_specs=specs,
        out_specs=[pl.BlockSpec((_NODE_BLK, d), row) for d in out_dims],
        out_shape=[jax.ShapeDtypeStruct((N_NODES, d), jnp.float32)
                   for d in out_dims],
    )(h, p0, p1, *ws)


@jax.jit
def _tc_node_mid(h, parts, node, wa, ba, wb):
    return _node_call(h, parts, node, [wa, ba, wb],
                      [N_HIDDEN, HP, HP], final=False)


@jax.jit
def _tc_node_final(h, parts, node, wo, bo):
    return _node_call(h, parts, node, [wo, bo], [D_FEAT], final=True)


def kernel(x, params, edge_index):
    src = edge_index[0].astype(jnp.int32)
    dst = edge_index[1].astype(jnp.int32)
    msg = params['msg']
    wm1, bm1 = msg[0]
    wma = _pad_cols(wm1[:N_HIDDEN])
    wmb = _pad_cols(wm1[N_HIDDEN:])
    bm1p = _pad_cols(bm1)
    (w2, b2), (w3, b3), (w4, b4) = msg[1], msg[2], msg[3]
    w2p = jnp.pad(w2, ((0, HP - N_HIDDEN), (0, 0)))
    w4p = _pad_cols(w4)
    b4p = _pad_cols(b4)
    wo, bo = params['out']

    h, a, b = _tc_pre(x, params['pre'], wma, bm1p, wmb)
    zeros = jnp.zeros((N_NODES, HP), jnp.float32)

    out = None
    for step in range(2):
        ea, eb = _sc_gather(a, b, src, dst)
        m = _tc_edge(ea, eb, w2p, b2, w3, b3, w4p, b4p)
        parts = _sc_scatter(m, dst, zeros)
        if step == 0:
            h, a, b = _tc_node_mid(h, parts, params['node'], wma, bm1p, wmb)
        else:
            (out,) = _tc_node_final(h, parts, params['node'], wo, bo)
    return out


# speedup vs baseline: 4.6921x; 1.2912x over previous
import jax
import jax.numpy as jnp
from jax import lax
from jax.experimental import pallas as pl
from jax.experimental.pallas import tpu as pltpu
from jax.experimental.pallas import tpu_sc as plsc

N_NODES = 10000
N_EDGES = 320000
D_FEAT = 128
N_HIDDEN = 96
HP = 128

_NC = 2
_NS = 16
_NW = _NC * _NS
_EPT = N_EDGES // _NW
_CH = 200
_NCHUNK = _EPT // _CH
_CHS = 80
_NCHUNKS = _EPT // _CHS
_ZB = 1000

_EDGE_BLK = 3200
_NODE_BLK = 2000


def _sc_mesh():
    return plsc.VectorSubcoreMesh(
        core_axis_name="c", subcore_axis_name="s",
        num_cores=_NC, num_subcores=_NS)


def _pad_cols(w):
    pad = [(0, 0)] * (w.ndim - 1) + [(0, HP - w.shape[-1])]
    return jnp.pad(w, pad)


def _gather_body(a_hbm, b_hbm, src_hbm, dst_hbm, e_hbm,
                 idx_s, idx_d, rows, sem_a, sem_b, sem_o):
    c = lax.axis_index("c")
    s = lax.axis_index("s")
    wid = s * _NC + c
    base0 = wid * _EPT

    pltpu.sync_copy(src_hbm.at[pl.ds(base0, _EPT)], idx_s)
    pltpu.sync_copy(dst_hbm.at[pl.ds(base0, _EPT)], idx_d)

    def start_a(k, p):
        @pl.when(k >= 2)
        def _():
            pltpu.make_async_copy(
                rows.at[p], e_hbm.at[pl.ds(base0 + (k - 2) * _CH, _CH)],
                sem_o.at[p]).wait()
        pltpu.async_copy(a_hbm.at[idx_s.at[pl.ds(k * _CH, _CH)]],
                         rows.at[p], sem_a.at[p])

    def start_b(k, p):
        pltpu.make_async_copy(a_hbm.at[idx_s.at[pl.ds(k * _CH, _CH)]],
                              rows.at[p], sem_a.at[p]).wait()
        pltpu.async_copy(b_hbm.at[idx_d.at[pl.ds(k * _CH, _CH)]],
                         rows.at[p], sem_b.at[p], add=True)

    def store(k, p):
        pltpu.make_async_copy(b_hbm.at[idx_d.at[pl.ds(k * _CH, _CH)]],
                              rows.at[p], sem_b.at[p]).wait()
        pltpu.async_copy(rows.at[p], e_hbm.at[pl.ds(base0 + k * _CH, _CH)],
                         sem_o.at[p])

    start_a(0, 0)

    def pair(i, carry):
        k0 = 2 * i
        k1 = k0 + 1
        start_a(k1, 1)
        start_b(k0, 0)
        store(k0, 0)

        @pl.when(i < _NCHUNK // 2 - 1)
        def _():
            start_a(k0 + 2, 0)
        start_b(k1, 1)
        store(k1, 1)
        return carry

    lax.fori_loop(0, _NCHUNK // 2, pair, 0)

    pltpu.make_async_copy(
        rows.at[0], e_hbm.at[pl.ds(base0 + (_NCHUNK - 2) * _CH, _CH)],
        sem_o.at[0]).wait()
    pltpu.make_async_copy(
        rows.at[1], e_hbm.at[pl.ds(base0 + (_NCHUNK - 1) * _CH, _CH)],
        sem_o.at[1]).wait()


@jax.jit
def _sc_gather(a, b, src2, dst2):
    f = pl.kernel(
        _gather_body,
        out_type=jax.ShapeDtypeStruct((N_EDGES, HP), jnp.float32),
        mesh=_sc_mesh(),
        scratch_types=[
            pltpu.VMEM((_EPT,), jnp.int32),
            pltpu.VMEM((_EPT,), jnp.int32),
            pltpu.VMEM((2, _CH, HP), jnp.float32),
            pltpu.SemaphoreType.DMA((2,)),
            pltpu.SemaphoreType.DMA((2,)),
            pltpu.SemaphoreType.DMA((2,)),
        ],
    )
    return f(a, b, src2, dst2)


def _scatter_body(m_hbm, dst_hbm, zeros_hbm, out_hbm,
                  idx, mbuf, acc, sem_m):
    c = lax.axis_index("c")
    s = lax.axis_index("s")
    wid = s * _NC + c
    base0 = wid * _EPT

    @pl.when(s < 10)
    def _zero():
        pltpu.sync_copy(zeros_hbm.at[pl.ds(s * _ZB, _ZB)],
                        acc.at[pl.ds(s * _ZB, _ZB)])
    pltpu.sync_copy(dst_hbm.at[wid], idx)
    plsc.subcore_barrier()

    def load(k, p):
        pltpu.async_copy(m_hbm.at[pl.ds(base0 + k * _CHS, _CHS)],
                         mbuf.at[p], sem_m.at[p])

    def add(k, p):
        pltpu.make_async_copy(m_hbm.at[pl.ds(base0 + k * _CHS, _CHS)],
                              mbuf.at[p], sem_m.at[p]).wait()
        pltpu.sync_copy(mbuf.at[p], acc.at[idx.at[k]], add=True)

    load(0, 0)

    def pair(i, carry):
        k0 = 2 * i
        k1 = k0 + 1
        load(k1, 1)
        add(k0, 0)
        load(k0 + 2, 0)
        add(k1, 1)
        return carry

    lax.fori_loop(0, _NCHUNKS // 2, pair, 0)
    add(_NCHUNKS - 1, 0)
    plsc.subcore_barrier()

    @pl.when(s < 10)
    def _writeback():
        pltpu.sync_copy(acc.at[pl.ds(s * _ZB, _ZB)],
                        out_hbm.at[pl.ds(c * N_NODES + s * _ZB, _ZB)])


@jax.jit
def _sc_scatter(m, dst2, zeros):
    f = pl.kernel(
        _scatter_body,
        out_type=jax.ShapeDtypeStruct((2 * N_NODES, HP), jnp.float32),
        mesh=_sc_mesh(),
        scratch_types=[
            pltpu.VMEM((_NCHUNKS, _CHS), jnp.int32),
            pltpu.VMEM((2, _CHS, HP), jnp.float32),
            pltpu.VMEM_SHARED((N_NODES, HP), jnp.float32),
            pltpu.SemaphoreType.DMA((2,)),
        ],
    )
    return f(m, dst2, zeros)


def _dot(x, w):
    return jax.lax.dot_general(x, w, (((1,), (0,)), ((), ())),
                               preferred_element_type=jnp.float32)


def _pre_body(x_ref, w0, b0, w1, b1, w2, b2, w3, b3, wa, ba, wb,
              h_ref, a_ref, bo_ref):
    h = jnp.maximum(_dot(x_ref[...], w0[...]) + b0[...], 0.0)
    h = jnp.maximum(_dot(h, w1[...]) + b1[...], 0.0)
    h = jnp.maximum(_dot(h, w2[...]) + b2[...], 0.0)
    h = _dot(h, w3[...]) + b3[...]
    h_ref[...] = h
    a_ref[...] = _dot(h, wa[...]) + ba[...]
    bo_ref[...] = _dot(h, wb[...])


@jax.jit
def _tc_pre(x, pre, wa, ba, wb):
    nblk = N_NODES // _NODE_BLK
    row = lambda i: (i, 0)
    cst = lambda i: (0, 0)
    ws = []
    specs = [pl.BlockSpec((_NODE_BLK, D_FEAT), row)]
    for (w, b) in pre:
        ws += [w, b.reshape(1, -1)]
        specs += [pl.BlockSpec(w.shape, cst), pl.BlockSpec((1, w.shape[1]), cst)]
    ws += [wa, ba.reshape(1, -1), wb]
    specs += [pl.BlockSpec(wa.shape, cst), pl.BlockSpec((1, HP), cst),
              pl.BlockSpec(wb.shape, cst)]
    return pl.pallas_call(
        _pre_body,
        grid=(nblk,),
        in_specs=specs,
        out_specs=[pl.BlockSpec((_NODE_BLK, N_HIDDEN), row),
                   pl.BlockSpec((_NODE_BLK, HP), row),
                   pl.BlockSpec((_NODE_BLK, HP), row)],
        out_shape=[jax.ShapeDtypeStruct((N_NODES, N_HIDDEN), jnp.float32),
                   jax.ShapeDtypeStruct((N_NODES, HP), jnp.float32),
                   jax.ShapeDtypeStruct((N_NODES, HP), jnp.float32)],
    )(x, *ws)


def _edge_body(e_ref, w2, b2, w3, b3, w4, b4, m_ref):
    e = jnp.maximum(e_ref[...], 0.0)
    e = jnp.maximum(_dot(e, w2[...]) + b2[...], 0.0)
    e = jnp.maximum(_dot(e, w3[...]) + b3[...], 0.0)
    m_ref[...] = _dot(e, w4[...]) + b4[...]


@jax.jit
def _tc_edge(e1, w2p, b2, w3, b3, w4p, b4p):
    nblk = N_EDGES // _EDGE_BLK
    row = lambda i: (i, 0)
    cst = lambda i: (0, 0)
    ws = [w2p, b2.reshape(1, -1), w3, b3.reshape(1, -1),
          w4p, b4p.reshape(1, -1)]
    specs = [pl.BlockSpec((_EDGE_BLK, HP), row)]
    for w in ws:
        specs.append(pl.BlockSpec(w.shape, cst))
    return pl.pallas_call(
        _edge_body,
        grid=(nblk,),
        in_specs=specs,
        out_specs=pl.BlockSpec((_EDGE_BLK, HP), row),
        out_shape=jax.ShapeDtypeStruct((N_EDGES, HP), jnp.float32),
    )(e1, *ws)


def _node_call(h, parts, node, heads, out_dims, final):
    p0 = parts[:N_NODES]
    p1 = parts[N_NODES:]
    nblk = N_NODES // _NODE_BLK
    row = lambda i: (i, 0)
    cst = lambda i: (0, 0)
    (wn1, bn1), n2, n3, n4 = node
    wna = wn1[:N_HIDDEN]
    wnb = jnp.pad(wn1[N_HIDDEN:], ((0, HP - N_HIDDEN), (0, 0)))
    ws = [wna, wnb, bn1.reshape(1, -1)]
    specs = [pl.BlockSpec((_NODE_BLK, N_HIDDEN), row),
             pl.BlockSpec((_NODE_BLK, HP), row),
             pl.BlockSpec((_NODE_BLK, HP), row)]
    specs += [pl.BlockSpec(wna.shape, cst), pl.BlockSpec(wnb.shape, cst),
              pl.BlockSpec((1, N_HIDDEN), cst)]
    for (w, b) in (n2, n3, n4):
        ws += [w, b.reshape(1, -1)]
        specs += [pl.BlockSpec(w.shape, cst), pl.BlockSpec((1, w.shape[1]), cst)]
    for hmat in heads:
        shp = hmat.shape if hmat.ndim == 2 else (1, hmat.shape[-1])
        ws.append(hmat.reshape(shp))
        specs.append(pl.BlockSpec(shp, cst))

    def body(h_ref, p0_ref, p1_ref, rwna, rwnb, rbn1, rw2, rb2, rw3, rb3,
             rw4, rb4, *rest):
        agg = p0_ref[...] + p1_ref[...]
        e = jnp.maximum(_dot(h_ref[...], rwna[...]) + _dot(agg, rwnb[...])
                        + rbn1[...], 0.0)
        e = jnp.maximum(_dot(e, rw2[...]) + rb2[...], 0.0)
        e = jnp.maximum(_dot(e, rw3[...]) + rb3[...], 0.0)
        hn = _dot(e, rw4[...]) + rb4[...]
        if final:
            rwo, rbo, out_ref = rest
            out_ref[...] = _dot(hn, rwo[...]) + rbo[...]
        else:
            rwa, rba, rwb, h_out, a_out, b_out = rest
            h_out[...] = hn
            a_out[...] = _dot(hn, rwa[...]) + rba[...]
            b_out[...] = _dot(hn, rwb[...])

    return pl.pallas_call(
        body,
        grid=(nblk,),
        in_specs=specs,
        out_specs=[pl.BlockSpec((_NODE_BLK, d), row) for d, _ in out_dims],
        out_shape=[jax.ShapeDtypeStruct((N_NODES, d), t)
                   for d, t in out_dims],
    )(h, p0, p1, *ws)


@jax.jit
def _tc_node_mid(h, parts, node, wa, ba, wb):
    return _node_call(h, parts, node, [wa, ba, wb],
                      [(N_HIDDEN, jnp.float32), (HP, jnp.float32),
                       (HP, jnp.float32)], final=False)


@jax.jit
def _tc_node_final(h, parts, node, wo, bo):
    return _node_call(h, parts, node, [wo, bo], [(D_FEAT, jnp.float32)],
                      final=True)


def kernel(x, params, edge_index):
    src = edge_index[0].astype(jnp.int32)
    dst = edge_index[1].astype(jnp.int32)
    dst_s = dst.reshape(_NW, _NCHUNKS, _CHS)
    msg = params['msg']
    wm1, bm1 = msg[0]
    wma = _pad_cols(wm1[:N_HIDDEN])
    wmb = _pad_cols(wm1[N_HIDDEN:])
    bm1p = _pad_cols(bm1)
    (w2, b2), (w3, b3), (w4, b4) = msg[1], msg[2], msg[3]
    w2p = jnp.pad(w2, ((0, HP - N_HIDDEN), (0, 0)))
    w4p = _pad_cols(w4)
    b4p = _pad_cols(b4)
    wo, bo = params['out']

    h, a, b = _tc_pre(x, params['pre'], wma, bm1p, wmb)
    zeros = jnp.zeros((N_NODES, HP), jnp.float32)

    out = None
    for step in range(2):
        e1 = _sc_gather(a, b, src, dst)
        m = _tc_edge(e1, w2p, b2, w3, b3, w4p, b4p)
        parts = _sc_scatter(m, dst_s, zeros)
        if step == 0:
            h, a, b = _tc_node_mid(h, parts, params['node'], wma, bm1p, wmb)
        else:
            (out,) = _tc_node_final(h, parts, params['node'], wo, bo)
    return out
